# baseline (speedup 1.0000x reference)
import functools

import jax
import jax.numpy as jnp
from jax import lax
from jax.experimental import pallas as pl
from jax.experimental.pallas import tpu as pltpu
from jax.experimental.pallas import tpu_sc as plsc

N = 10000
E = 320000
NC, NS = 2, 16
NW = NC * NS
EPW = E // NW
C = 80
NCHUNK = EPW // C
N2 = 10240
RPS = N2 // NS
DPS = N2 // NS


def _make_sc_agg(D, want_deg):
  out_type = [jax.ShapeDtypeStruct((2 * N2, D), jnp.float32)]
  scratch = [
      pltpu.VMEM((EPW,), jnp.int32),
      pltpu.VMEM((NCHUNK, C), jnp.int32),
      pltpu.VMEM((C, D), jnp.float32),
      pltpu.VMEM((C, D), jnp.float32),
      pltpu.VMEM_SHARED((N2, D), jnp.float32),
      pltpu.SemaphoreType.DMA,
      pltpu.SemaphoreType.DMA,
  ]
  if want_deg:
    out_type.append(jax.ShapeDtypeStruct((2 * N2,), jnp.float32))
    scratch += [
        pltpu.VMEM((C,), jnp.float32),
        pltpu.VMEM((C,), jnp.float32),
        pltpu.VMEM_SHARED((N2,), jnp.float32),
    ]

  mesh = plsc.VectorSubcoreMesh(core_axis_name="c", subcore_axis_name="s")
  params = (pltpu.CompilerParams(use_tc_tiling_on_sc=False)
            if D % 128 else None)

  @functools.partial(pl.kernel, out_type=out_type, mesh=mesh,
                     scratch_types=scratch, compiler_params=params)
  def sc_agg(p_hbm, src_hbm, dst_hbm, out_hbm, *rest):
    if want_deg:
      (deg_hbm, src_v, dst_v, rows_a, rows_b, acc, gsa, gsb,
       dzbuf, ones_v, dacc) = rest
    else:
      src_v, dst_v, rows_a, rows_b, acc, gsa, gsb = rest
    rows_v = rows_a
    c = lax.axis_index("c")
    s = lax.axis_index("s")
    w = c * NS + s

    pltpu.async_copy(src_hbm.at[w], src_v, gsa)
    pltpu.async_copy(dst_hbm.at[w], dst_v, gsb)

    z16 = jnp.zeros((16,), jnp.float32)

    def zrow(i, carry):
      for k in range(D // 16):
        rows_v[i, pl.ds(k * 16, 16)] = z16
      return carry

    lax.fori_loop(0, C, zrow, 0)
    for r in range(RPS // C):
      pltpu.sync_copy(rows_v, acc.at[pl.ds(s * RPS + r * C, C)])

    if want_deg:
      one16 = jnp.ones((16,), jnp.float32)
      for k in range(C // 16):
        dzbuf[pl.ds(k * 16, 16)] = z16
        ones_v[pl.ds(k * 16, 16)] = one16
      for r in range(DPS // C):
        pltpu.sync_copy(dzbuf, dacc.at[pl.ds(s * DPS + r * C, C)])

    pltpu.make_async_copy(src_hbm.at[w], src_v, gsa).wait()
    pltpu.make_async_copy(dst_hbm.at[w], dst_v, gsb).wait()
    plsc.subcore_barrier()

    def g_start(j, buf, sem):
      pltpu.async_copy(p_hbm.at[src_v.at[pl.ds(j * C, C)]], buf, sem)

    def g_wait(j, buf, sem):
      pltpu.make_async_copy(p_hbm.at[src_v.at[pl.ds(j * C, C)]],
                            buf, sem).wait()

    def consume(j, buf):
      pltpu.sync_copy(buf, acc.at[dst_v.at[j]], add=True)
      if want_deg:
        pltpu.sync_copy(ones_v, dacc.at[dst_v.at[j]], add=True)

    g_start(0, rows_a, gsa)
    g_start(1, rows_b, gsb)

    def pair(t, carry):
      ja = 2 * t
      jb = 2 * t + 1
      g_wait(ja, rows_a, gsa)
      consume(ja, rows_a)
      g_start(ja + 2, rows_a, gsa)
      g_wait(jb, rows_b, gsb)
      consume(jb, rows_b)

      @pl.when(t < (NCHUNK - 1) // 2 - 1)
      def _():
        g_start(jb + 2, rows_b, gsb)

      return carry

    lax.fori_loop(0, (NCHUNK - 1) // 2, pair, 0)
    jl = NCHUNK - 1
    g_wait(jl, rows_a, gsa)
    consume(jl, rows_a)
    plsc.subcore_barrier()

    pltpu.sync_copy(acc.at[pl.ds(s * RPS, RPS)],
                    out_hbm.at[pl.ds(c * N2 + s * RPS, RPS)])
    if want_deg:
      pltpu.sync_copy(dacc.at[pl.ds(s * DPS, DPS)],
                      deg_hbm.at[pl.ds(c * N2 + s * DPS, DPS)])

  return sc_agg


_sc_agg_deg = _make_sc_agg(128, True)
_sc_agg128 = _make_sc_agg(128, False)
_sc_agg64 = _make_sc_agg(64, False)

BM = 2000


def _proj_body(x_ref, w_ref, o_ref):
  o_ref[...] = jnp.dot(x_ref[...], w_ref[...],
                       preferred_element_type=jnp.float32)


def _tc_proj(x, w):
  din, dout = w.shape
  return pl.pallas_call(
      _proj_body,
      grid=(N // BM,),
      in_specs=[pl.BlockSpec((BM, din), lambda i: (i, 0)),
                pl.BlockSpec((din, dout), lambda i: (0, 0))],
      out_specs=pl.BlockSpec((BM, dout), lambda i: (i, 0)),
      out_shape=jax.ShapeDtypeStruct((N, dout), jnp.float32),
  )(x, w)


def _combine_body(relu, project, dl, hs_ref, p0_ref, p1_ref, deg_ref,
                  b_ref, *rest):
  if project:
    wn_ref, hn_ref, pn_ref = rest
  else:
    hn_ref, = rest
  deg = deg_ref[...]
  tot = deg[:, 0:1] + deg[:, 1:2]
  scale = 1.0 / jnp.maximum(tot, 1.0)
  mean = (p0_ref[:, :dl] + p1_ref[:, :dl]) * scale
  hn = hs_ref[...] + mean + b_ref[...]
  if relu:
    hn = jnp.maximum(hn, 0.0)
  hn_ref[...] = hn
  if project:
    pn_ref[...] = jnp.dot(hn, wn_ref[...],
                          preferred_element_type=jnp.float32)


def _tc_combine(hs, part0, part1, deg2, b, wn=None, relu=True):
  dl = hs.shape[1]
  pw = part0.shape[1]
  in_specs = [
      pl.BlockSpec((BM, dl), lambda i: (i, 0)),
      pl.BlockSpec((BM, pw), lambda i: (i, 0)),
      pl.BlockSpec((BM, pw), lambda i: (i, 0)),
      pl.BlockSpec((BM, 2), lambda i: (i, 0)),
      pl.BlockSpec((1, dl), lambda i: (0, 0)),
  ]
  out_shape = [jax.ShapeDtypeStruct((N, dl), jnp.float32)]
  out_specs = [pl.BlockSpec((BM, dl), lambda i: (i, 0))]
  args = [hs, part0, part1, deg2, b.reshape(1, dl)]
  project = wn is not None
  if project:
    dnext = wn.shape[1]
    in_specs.append(pl.BlockSpec((dl, dnext), lambda i: (0, 0)))
    out_shape.append(jax.ShapeDtypeStruct((N, dnext), jnp.float32))
    out_specs.append(pl.BlockSpec((BM, dnext), lambda i: (i, 0)))
    args.append(wn)
  outs = pl.pallas_call(
      functools.partial(_combine_body, relu, project, dl),
      grid=(N // BM,),
      in_specs=in_specs,
      out_specs=out_specs,
      out_shape=out_shape,
  )(*args)
  return outs if project else outs[0]


def kernel(x, edge_index, W_self0, W_neigh0, b0, W_self1, W_neigh1, b1,
           W_self2, W_neigh2, b2):
  src = edge_index[0].astype(jnp.int32).reshape(NW, EPW)
  dst = edge_index[1].astype(jnp.int32).reshape(NW, NCHUNK, C)

  p0 = _tc_proj(x, W_neigh0)
  hs0 = _tc_proj(x, W_self0)
  agg0, deg_raw = _sc_agg_deg(p0, src, dst)
  deg2 = deg_raw.reshape(2, N2)[:, :N].T

  h1, p1 = _tc_combine(hs0, agg0[:N], agg0[N2:N2 + N], deg2, b0,
                       wn=W_neigh1)
  hs1 = _tc_proj(h1, W_self1)
  agg1, = _sc_agg128(p1, src, dst)
  h2, p2 = _tc_combine(hs1, agg1[:N], agg1[N2:N2 + N], deg2, b1,
                       wn=W_neigh2)
  hs2 = _tc_proj(h2, W_self2)
  agg2, = _sc_agg64(p2, src, dst)
  out = _tc_combine(hs2, agg2[:N], agg2[N2:N2 + N], deg2, b2,
                    relu=False)
  return out


# speedup vs baseline: 1.0044x; 1.0044x over previous
import functools

import jax
import jax.numpy as jnp
from jax import lax
from jax.experimental import pallas as pl
from jax.experimental.pallas import tpu as pltpu
from jax.experimental.pallas import tpu_sc as plsc

N = 10000
E = 320000
NC, NS = 2, 16
NW = NC * NS
EPW = E // NW
C = 80
NCHUNK = EPW // C
N2 = 10240
RPS = N2 // NS
DPS = N2 // NS


def _make_sc_agg(D, want_deg):
  out_type = [jax.ShapeDtypeStruct((2 * N2, D), jnp.float32)]
  scratch = [
      pltpu.VMEM((EPW,), jnp.int32),
      pltpu.VMEM((NCHUNK, C), jnp.int32),
      pltpu.VMEM((C, D), jnp.float32),
      pltpu.VMEM((C, D), jnp.float32),
      pltpu.VMEM_SHARED((N2, D), jnp.float32),
      pltpu.SemaphoreType.DMA,
      pltpu.SemaphoreType.DMA,
  ]
  if want_deg:
    out_type.append(jax.ShapeDtypeStruct((2 * N2,), jnp.float32))
    scratch += [
        pltpu.VMEM((C,), jnp.float32),
        pltpu.VMEM((C,), jnp.float32),
        pltpu.VMEM_SHARED((N2,), jnp.float32),
    ]

  mesh = plsc.VectorSubcoreMesh(core_axis_name="c", subcore_axis_name="s")
  params = (pltpu.CompilerParams(use_tc_tiling_on_sc=False)
            if D % 128 else None)

  @functools.partial(pl.kernel, out_type=out_type, mesh=mesh,
                     scratch_types=scratch, compiler_params=params)
  def sc_agg(p_hbm, src_hbm, dst_hbm, out_hbm, *rest):
    if want_deg:
      (deg_hbm, src_v, dst_v, rows_a, rows_b, acc, gsa, gsb,
       dzbuf, ones_v, dacc) = rest
    else:
      src_v, dst_v, rows_a, rows_b, acc, gsa, gsb = rest
    rows_v = rows_a
    c = lax.axis_index("c")
    s = lax.axis_index("s")
    w = c * NS + s

    pltpu.async_copy(src_hbm.at[w], src_v, gsa)
    pltpu.async_copy(dst_hbm.at[w], dst_v, gsb)

    z16 = jnp.zeros((16,), jnp.float32)

    def zrow(i, carry):
      for k in range(D // 16):
        rows_v[i, pl.ds(k * 16, 16)] = z16
      return carry

    lax.fori_loop(0, C, zrow, 0)
    for r in range(RPS // C):
      pltpu.sync_copy(rows_v, acc.at[pl.ds(s * RPS + r * C, C)])

    if want_deg:
      one16 = jnp.ones((16,), jnp.float32)
      for k in range(C // 16):
        dzbuf[pl.ds(k * 16, 16)] = z16
        ones_v[pl.ds(k * 16, 16)] = one16
      for r in range(DPS // C):
        pltpu.sync_copy(dzbuf, dacc.at[pl.ds(s * DPS + r * C, C)])

    pltpu.make_async_copy(src_hbm.at[w], src_v, gsa).wait()
    pltpu.make_async_copy(dst_hbm.at[w], dst_v, gsb).wait()
    plsc.subcore_barrier()

    def g_start(j, buf, sem):
      pltpu.async_copy(p_hbm.at[src_v.at[pl.ds(j * C, C)]], buf, sem)

    def g_wait(j, buf, sem):
      pltpu.make_async_copy(p_hbm.at[src_v.at[pl.ds(j * C, C)]],
                            buf, sem).wait()

    def consume(j, buf):
      pltpu.sync_copy(buf, acc.at[dst_v.at[j]], add=True)
      if want_deg:
        pltpu.sync_copy(ones_v, dacc.at[dst_v.at[j]], add=True)

    g_start(0, rows_a, gsa)
    g_start(1, rows_b, gsb)

    def pair(t, carry):
      ja = 2 * t
      jb = 2 * t + 1
      g_wait(ja, rows_a, gsa)
      consume(ja, rows_a)
      g_start(ja + 2, rows_a, gsa)
      g_wait(jb, rows_b, gsb)
      consume(jb, rows_b)

      @pl.when(t < (NCHUNK - 1) // 2 - 1)
      def _():
        g_start(jb + 2, rows_b, gsb)

      return carry

    lax.fori_loop(0, (NCHUNK - 1) // 2, pair, 0)
    jl = NCHUNK - 1
    g_wait(jl, rows_a, gsa)
    consume(jl, rows_a)
    plsc.subcore_barrier()

    pltpu.sync_copy(acc.at[pl.ds(s * RPS, RPS)],
                    out_hbm.at[pl.ds(c * N2 + s * RPS, RPS)])
    if want_deg:
      pltpu.sync_copy(dacc.at[pl.ds(s * DPS, DPS)],
                      deg_hbm.at[pl.ds(c * N2 + s * DPS, DPS)])

  return sc_agg


_sc_agg_deg = _make_sc_agg(128, True)
_sc_agg128 = _make_sc_agg(128, False)
_sc_agg64 = _make_sc_agg(64, False)

BM = 2000


def _proj_body(x_ref, w_ref, o_ref):
  o_ref[...] = jnp.dot(x_ref[...], w_ref[...],
                       preferred_element_type=jnp.float32)


def _tc_proj(x, w):
  din, dout = w.shape
  return pl.pallas_call(
      _proj_body,
      grid=(N // BM,),
      in_specs=[pl.BlockSpec((BM, din), lambda i: (i, 0)),
                pl.BlockSpec((din, dout), lambda i: (0, 0))],
      out_specs=pl.BlockSpec((BM, dout), lambda i: (i, 0)),
      out_shape=jax.ShapeDtypeStruct((N, dout), jnp.float32),
  )(x, w)


def _combine_body(relu, project, dl, h_ref, p0_ref, p1_ref, deg_ref, ws_ref,
                  b_ref, *rest):
  if project:
    wn_ref, hn_ref, pn_ref = rest
  else:
    hn_ref, = rest
  deg = deg_ref[...]
  tot = deg[:, 0:1] + deg[:, 1:2]
  scale = 1.0 / jnp.maximum(tot, 1.0)
  mean = (p0_ref[:, :dl] + p1_ref[:, :dl]) * scale
  hn = (jnp.dot(h_ref[...], ws_ref[...], preferred_element_type=jnp.float32)
        + mean + b_ref[...])
  if relu:
    hn = jnp.maximum(hn, 0.0)
  hn_ref[...] = hn
  if project:
    pn_ref[...] = jnp.dot(hn, wn_ref[...],
                          preferred_element_type=jnp.float32)


def _tc_combine(h, part0, part1, deg2, ws, b, wn=None, relu=True):
  din, dl = ws.shape
  pw = part0.shape[1]
  in_specs = [
      pl.BlockSpec((BM, din), lambda i: (i, 0)),
      pl.BlockSpec((BM, pw), lambda i: (i, 0)),
      pl.BlockSpec((BM, pw), lambda i: (i, 0)),
      pl.BlockSpec((BM, 2), lambda i: (i, 0)),
      pl.BlockSpec((din, dl), lambda i: (0, 0)),
      pl.BlockSpec((1, dl), lambda i: (0, 0)),
  ]
  out_shape = [jax.ShapeDtypeStruct((N, dl), jnp.float32)]
  out_specs = [pl.BlockSpec((BM, dl), lambda i: (i, 0))]
  args = [h, part0, part1, deg2, ws, b.reshape(1, dl)]
  project = wn is not None
  if project:
    dnext = wn.shape[1]
    in_specs.append(pl.BlockSpec((dl, dnext), lambda i: (0, 0)))
    out_shape.append(jax.ShapeDtypeStruct((N, dnext), jnp.float32))
    out_specs.append(pl.BlockSpec((BM, dnext), lambda i: (i, 0)))
    args.append(wn)
  outs = pl.pallas_call(
      functools.partial(_combine_body, relu, project, dl),
      grid=(N // BM,),
      in_specs=in_specs,
      out_specs=out_specs,
      out_shape=out_shape,
  )(*args)
  return outs if project else outs[0]


def kernel(x, edge_index, W_self0, W_neigh0, b0, W_self1, W_neigh1, b1,
           W_self2, W_neigh2, b2):
  src = edge_index[0].astype(jnp.int32).reshape(NW, EPW)
  dst = edge_index[1].astype(jnp.int32).reshape(NW, NCHUNK, C)

  p0 = _tc_proj(x, W_neigh0)
  agg0, deg_raw = _sc_agg_deg(p0, src, dst)
  deg2 = deg_raw.reshape(2, N2)[:, :N].T

  h1, p1 = _tc_combine(x, agg0[:N], agg0[N2:N2 + N], deg2, W_self0, b0,
                       wn=W_neigh1)
  agg1, = _sc_agg128(p1, src, dst)
  h2, p2 = _tc_combine(h1, agg1[:N], agg1[N2:N2 + N], deg2, W_self1, b1,
                       wn=W_neigh2)
  agg2, = _sc_agg64(p2, src, dst)
  out = _tc_combine(h2, agg2[:N], agg2[N2:N2 + N], deg2, W_self2, b2,
                    relu=False)
  return out
